# baseline (speedup 1.0000x reference)
import jax
import jax.numpy as jnp
from jax import lax
from jax.experimental import pallas as pl
from jax.experimental.pallas import tpu as pltpu
from jax.experimental.pallas import tpu_sc as plsc

_N_BITS = 20
_N_OPS = 64
_NUM_SAMPLES = 1048576
_NUM_BINS = 1 << _N_BITS
_NH = 1024
_NL = 1024
_NB = 16


_BLKL = 131072
_HALF = _NUM_SAMPLES // 2
_GRID_H = _HALF // _BLKL


def _codes_body(samples_ref, codes_ref):
    x = samples_ref[...].astype(jnp.bfloat16)
    powers = jnp.exp2(
        lax.broadcasted_iota(jnp.int32, (1, _N_BITS), 1)
        .astype(jnp.float32)).astype(jnp.bfloat16)
    codesf = lax.dot_general(powers, x, (((1,), (0,)), ((), ())),
                             preferred_element_type=jnp.float32)
    codes_ref[...] = jnp.round(codesf.reshape(_BLKL)).astype(jnp.int32)


def _codes_stage(samples_t, off):
    return pl.pallas_call(
        _codes_body,
        grid=(_GRID_H,),
        in_specs=[pl.BlockSpec((_N_BITS, _BLKL), lambda i: (0, i + off))],
        out_specs=pl.BlockSpec((_BLKL,), lambda i: (i,)),
        out_shape=jax.ShapeDtypeStruct((_HALF,), jnp.int32),
    )(samples_t)


_N_TILES = 16
_N_CORES = 2
_PER_TILE = _HALF // (_N_TILES * _N_CORES)
_IDX_ROWS = _PER_TILE // 128
_CHUNK_ROWS = 64
_N_CHUNKS = _IDX_ROWS // _CHUNK_ROWS
_BINS_PER_TILE = _NUM_BINS // _N_TILES
_ZCHUNK = 2048


def _sc_hist_body(codes_hbm, out_hbm, table, idx_v, ones_v, zeros_v):
    cid = lax.axis_index("c")
    sid = lax.axis_index("s")

    def fillz(i, _):
        zeros_v[pl.ds(i * 16, 16)] = jnp.zeros((16,), jnp.int32)
        return 0
    lax.fori_loop(0, _ZCHUNK // 16, fillz, 0)

    def fill1(i, _):
        ones_v[pl.ds(i * 16, 16)] = jnp.ones((16,), jnp.int32)
        return 0
    lax.fori_loop(0, 128 // 16, fill1, 0)

    def zslice(j, _):
        pltpu.sync_copy(
            zeros_v,
            table.at[pl.ds(sid * _BINS_PER_TILE + j * _ZCHUNK, _ZCHUNK)])
        return 0
    lax.fori_loop(0, _BINS_PER_TILE // _ZCHUNK, zslice, 0)
    plsc.subcore_barrier()

    row0 = (cid * _N_TILES + sid) * _IDX_ROWS

    def chunk(c, _):
        pltpu.sync_copy(
            codes_hbm.at[pl.ds(row0 + c * _CHUNK_ROWS, _CHUNK_ROWS)], idx_v)

        def scat(j, _):
            pltpu.sync_copy(ones_v, table.at[idx_v.at[j]], add=True)
            return 0
        lax.fori_loop(0, _CHUNK_ROWS, scat, 0)
        return 0
    lax.fori_loop(0, _N_CHUNKS, chunk, 0)
    plsc.subcore_barrier()

    pltpu.sync_copy(
        table.at[pl.ds(sid * _BINS_PER_TILE, _BINS_PER_TILE)],
        out_hbm.at[pl.ds(cid * _NUM_BINS + sid * _BINS_PER_TILE,
                         _BINS_PER_TILE)])


def _sc_hist(codes):
    k = pl.kernel(
        _sc_hist_body,
        out_type=jax.ShapeDtypeStruct((_N_CORES * _NUM_BINS,), jnp.int32),
        mesh=plsc.VectorSubcoreMesh(core_axis_name="c", subcore_axis_name="s"),
        scratch_types=[
            pltpu.VMEM_SHARED((_NUM_BINS,), jnp.int32),
            pltpu.VMEM((_CHUNK_ROWS, 128), jnp.int32),
            pltpu.VMEM((128,), jnp.int32),
            pltpu.VMEM((_ZCHUNK,), jnp.int32),
        ],
    )
    return k(codes.reshape(_HALF // 128, 128))


_NR = 8192
_NC = 128
_RB = 13
_CB = 7


def _bins_body(pa_ref, pb_ref, mcr_ref, arr_ref, acc_ref, obsr_ref, obsc_ref,
               counts_ref, exp_ref):
    counts = (pa_ref[0, :, :] + pa_ref[1, :, :]
              + pb_ref[0, :, :] + pb_ref[1, :, :])
    counts_ref[...] = counts

    xr = ((lax.broadcasted_iota(jnp.int32, (_NR, _NB), 0)
           >> lax.broadcasted_iota(jnp.int32, (_NR, _NB), 1)) & 1
          ).astype(jnp.float32)
    xc = ((lax.broadcasted_iota(jnp.int32, (_NB, _NC), 1)
           >> lax.broadcasted_iota(jnp.int32, (_NB, _NC), 0)) & 1
          ).astype(jnp.float32)

    dn = (((1,), (0,)), ((), ()))
    f32 = jnp.float32
    g = lax.dot_general(xr, arr_ref[...], dn, preferred_element_type=f32)
    qrow = lax.dot_general(g * xr, jnp.ones((_NB, 1), f32), dn,
                           preferred_element_type=f32)
    b = lax.dot_general(acc_ref[...], xc, dn, preferred_element_type=f32)
    qlane = lax.dot_general(jnp.ones((1, _NB), f32), b * xc, dn,
                            preferred_element_type=f32)
    xm = lax.dot_general(xr, mcr_ref[...], dn, preferred_element_type=f32)
    xmext = jnp.concatenate([xm, qrow, jnp.ones((_NR, 1), f32)], axis=1)
    xcext = jnp.concatenate([xc, jnp.ones((1, _NC), f32), qlane], axis=0)
    t = lax.dot_general(xmext, xcext, dn, preferred_element_type=f32)

    mx = jnp.max(t)
    w = counts.astype(f32) * jnp.exp(t - mx)

    rowsum = lax.dot_general(w, jnp.ones((_NC, 1), f32), dn,
                             preferred_element_type=f32)
    colsum = lax.dot_general(jnp.ones((1, _NR), f32), w, dn,
                             preferred_element_type=f32)
    s = jnp.sum(rowsum)
    vr = lax.dot_general(rowsum, xr, (((0,), (0,)), ((), ())),
                         preferred_element_type=f32)
    vc = lax.dot_general(colsum, xc, (((1,), (1,)), ((), ())),
                         preferred_element_type=f32)

    zr = 1.0 - 2.0 * vr / s
    zc = 1.0 - 2.0 * vc / s
    dnr = (((1,), (1,)), ((), ()))
    er = lax.dot_general(obsr_ref[...], zr, dnr,
                         preferred_element_type=f32)
    ec = lax.dot_general(obsc_ref[...], zc, dnr, preferred_element_type=f32)
    exp_ref[...] = er + ec


def _bins_stage(pa, pb, mcr, arr, acc, obsr, obsc):
    return pl.pallas_call(
        _bins_body,
        grid=(1,),
        in_specs=[
            pl.BlockSpec((_N_CORES, _NR, _NC), lambda i: (0, 0, 0)),
            pl.BlockSpec((_N_CORES, _NR, _NC), lambda i: (0, 0, 0)),
            pl.BlockSpec((_NB, _NB), lambda i: (0, 0)),
            pl.BlockSpec((_NB, _NB), lambda i: (0, 0)),
            pl.BlockSpec((_NB, _NB), lambda i: (0, 0)),
            pl.BlockSpec((_N_OPS, _NB), lambda i: (0, 0)),
            pl.BlockSpec((_N_OPS, _NB), lambda i: (0, 0)),
        ],
        out_specs=[
            pl.BlockSpec((_NR, _NC), lambda i: (0, 0)),
            pl.BlockSpec((_N_OPS, 1), lambda i: (0, 0)),
        ],
        out_shape=[
            jax.ShapeDtypeStruct((_NR, _NC), jnp.int32),
            jax.ShapeDtypeStruct((_N_OPS, 1), jnp.float32),
        ],
    )(pa, pb, mcr, arr, acc, obsr, obsc)


def kernel(samples, theta, kernel, observables):
    samples_t = samples.T
    codes_a = _codes_stage(samples_t, 0)
    codes_b = _codes_stage(samples_t, _GRID_H)
    pa = _sc_hist(codes_a)
    pb = _sc_hist(codes_b)

    a = kernel + jnp.diag(theta)
    nb, cb = _NB, _CB
    mcr = jnp.zeros((nb, nb), jnp.float32).at[:_RB, :cb].set(
        a[cb:, :cb] + a[:cb, cb:].T)
    arr = jnp.zeros((nb, nb), jnp.float32).at[:_RB, :_RB].set(a[cb:, cb:])
    acc = jnp.zeros((nb, nb), jnp.float32).at[:cb, :cb].set(a[:cb, :cb])
    obsr = jnp.zeros((_N_OPS, nb), jnp.float32).at[:, :_RB].set(
        observables[:, cb:])
    obsc = jnp.zeros((_N_OPS, nb), jnp.float32).at[:, :cb].set(
        observables[:, :cb])

    counts2d, exps = _bins_stage(pa.reshape(_N_CORES, _NR, _NC),
                                 pb.reshape(_N_CORES, _NR, _NC),
                                 mcr, arr, acc, obsr, obsc)
    return counts2d.reshape(_NUM_BINS), exps.reshape(_N_OPS)


# speedup vs baseline: 1.0053x; 1.0053x over previous
import jax
import jax.numpy as jnp
from jax import lax
from jax.experimental import pallas as pl
from jax.experimental.pallas import tpu as pltpu
from jax.experimental.pallas import tpu_sc as plsc

_N_BITS = 20
_N_OPS = 64
_NUM_SAMPLES = 1048576
_NUM_BINS = 1 << _N_BITS
_NH = 1024
_NL = 1024
_NB = 16


_BLKL = 65536
_HALF = _NUM_SAMPLES // 2
_GRID_H = _HALF // _BLKL


def _codes_body(samples_ref, codes_ref):
    x = samples_ref[...].astype(jnp.bfloat16)
    powers = jnp.exp2(
        lax.broadcasted_iota(jnp.int32, (1, _N_BITS), 1)
        .astype(jnp.float32)).astype(jnp.bfloat16)
    codesf = lax.dot_general(powers, x, (((1,), (0,)), ((), ())),
                             preferred_element_type=jnp.float32)
    codes_ref[...] = jnp.round(codesf.reshape(_BLKL)).astype(jnp.int32)


def _codes_stage(samples_t, off):
    return pl.pallas_call(
        _codes_body,
        grid=(_GRID_H,),
        in_specs=[pl.BlockSpec((_N_BITS, _BLKL), lambda i: (0, i + off))],
        out_specs=pl.BlockSpec((_BLKL,), lambda i: (i,)),
        out_shape=jax.ShapeDtypeStruct((_HALF,), jnp.int32),
    )(samples_t)


_N_TILES = 16
_N_CORES = 2
_PER_TILE = _HALF // (_N_TILES * _N_CORES)
_IDX_ROWS = _PER_TILE // 128
_CHUNK_ROWS = 64
_N_CHUNKS = _IDX_ROWS // _CHUNK_ROWS
_BINS_PER_TILE = _NUM_BINS // _N_TILES
_ZCHUNK = 2048


def _sc_hist_body(codes_hbm, out_hbm, table, idx_v, ones_v, zeros_v):
    cid = lax.axis_index("c")
    sid = lax.axis_index("s")

    def fillz(i, _):
        zeros_v[pl.ds(i * 16, 16)] = jnp.zeros((16,), jnp.int32)
        return 0
    lax.fori_loop(0, _ZCHUNK // 16, fillz, 0)

    def fill1(i, _):
        ones_v[pl.ds(i * 16, 16)] = jnp.ones((16,), jnp.int32)
        return 0
    lax.fori_loop(0, 128 // 16, fill1, 0)

    def zslice(j, _):
        pltpu.sync_copy(
            zeros_v,
            table.at[pl.ds(sid * _BINS_PER_TILE + j * _ZCHUNK, _ZCHUNK)])
        return 0
    lax.fori_loop(0, _BINS_PER_TILE // _ZCHUNK, zslice, 0)
    plsc.subcore_barrier()

    row0 = (cid * _N_TILES + sid) * _IDX_ROWS

    def chunk(c, _):
        pltpu.sync_copy(
            codes_hbm.at[pl.ds(row0 + c * _CHUNK_ROWS, _CHUNK_ROWS)], idx_v)

        def scat(j, _):
            pltpu.sync_copy(ones_v, table.at[idx_v.at[j]], add=True)
            return 0
        lax.fori_loop(0, _CHUNK_ROWS, scat, 0)
        return 0
    lax.fori_loop(0, _N_CHUNKS, chunk, 0)
    plsc.subcore_barrier()

    pltpu.sync_copy(
        table.at[pl.ds(sid * _BINS_PER_TILE, _BINS_PER_TILE)],
        out_hbm.at[pl.ds(cid * _NUM_BINS + sid * _BINS_PER_TILE,
                         _BINS_PER_TILE)])


def _sc_hist(codes):
    k = pl.kernel(
        _sc_hist_body,
        out_type=jax.ShapeDtypeStruct((_N_CORES * _NUM_BINS,), jnp.int32),
        mesh=plsc.VectorSubcoreMesh(core_axis_name="c", subcore_axis_name="s"),
        scratch_types=[
            pltpu.VMEM_SHARED((_NUM_BINS,), jnp.int32),
            pltpu.VMEM((_CHUNK_ROWS, 128), jnp.int32),
            pltpu.VMEM((128,), jnp.int32),
            pltpu.VMEM((_ZCHUNK,), jnp.int32),
        ],
    )
    return k(codes.reshape(_HALF // 128, 128))


_NR = 8192
_NC = 128
_RB = 13
_CB = 7


def _bins_body(pa_ref, pb_ref, mcr_ref, arr_ref, acc_ref, obsr_ref, obsc_ref,
               counts_ref, exp_ref):
    counts = (pa_ref[0, :, :] + pa_ref[1, :, :]
              + pb_ref[0, :, :] + pb_ref[1, :, :])
    counts_ref[...] = counts

    xr = ((lax.broadcasted_iota(jnp.int32, (_NR, _NB), 0)
           >> lax.broadcasted_iota(jnp.int32, (_NR, _NB), 1)) & 1
          ).astype(jnp.float32)
    xc = ((lax.broadcasted_iota(jnp.int32, (_NB, _NC), 1)
           >> lax.broadcasted_iota(jnp.int32, (_NB, _NC), 0)) & 1
          ).astype(jnp.float32)

    dn = (((1,), (0,)), ((), ()))
    f32 = jnp.float32
    g = lax.dot_general(xr, arr_ref[...], dn, preferred_element_type=f32)
    qrow = lax.dot_general(g * xr, jnp.ones((_NB, 1), f32), dn,
                           preferred_element_type=f32)
    b = lax.dot_general(acc_ref[...], xc, dn, preferred_element_type=f32)
    qlane = lax.dot_general(jnp.ones((1, _NB), f32), b * xc, dn,
                            preferred_element_type=f32)
    xm = lax.dot_general(xr, mcr_ref[...], dn, preferred_element_type=f32)
    xmext = jnp.concatenate([xm, qrow, jnp.ones((_NR, 1), f32)], axis=1)
    xcext = jnp.concatenate([xc, jnp.ones((1, _NC), f32), qlane], axis=0)
    t = lax.dot_general(xmext, xcext, dn, preferred_element_type=f32)

    mx = jnp.max(t)
    w = counts.astype(f32) * jnp.exp(t - mx)

    rowsum = lax.dot_general(w, jnp.ones((_NC, 1), f32), dn,
                             preferred_element_type=f32)
    colsum = lax.dot_general(jnp.ones((1, _NR), f32), w, dn,
                             preferred_element_type=f32)
    s = jnp.sum(rowsum)
    vr = lax.dot_general(rowsum, xr, (((0,), (0,)), ((), ())),
                         preferred_element_type=f32)
    vc = lax.dot_general(colsum, xc, (((1,), (1,)), ((), ())),
                         preferred_element_type=f32)

    zr = 1.0 - 2.0 * vr / s
    zc = 1.0 - 2.0 * vc / s
    dnr = (((1,), (1,)), ((), ()))
    er = lax.dot_general(obsr_ref[...], zr, dnr,
                         preferred_element_type=f32)
    ec = lax.dot_general(obsc_ref[...], zc, dnr, preferred_element_type=f32)
    exp_ref[...] = er + ec


def _bins_stage(pa, pb, mcr, arr, acc, obsr, obsc):
    return pl.pallas_call(
        _bins_body,
        grid=(1,),
        in_specs=[
            pl.BlockSpec((_N_CORES, _NR, _NC), lambda i: (0, 0, 0)),
            pl.BlockSpec((_N_CORES, _NR, _NC), lambda i: (0, 0, 0)),
            pl.BlockSpec((_NB, _NB), lambda i: (0, 0)),
            pl.BlockSpec((_NB, _NB), lambda i: (0, 0)),
            pl.BlockSpec((_NB, _NB), lambda i: (0, 0)),
            pl.BlockSpec((_N_OPS, _NB), lambda i: (0, 0)),
            pl.BlockSpec((_N_OPS, _NB), lambda i: (0, 0)),
        ],
        out_specs=[
            pl.BlockSpec((_NR, _NC), lambda i: (0, 0)),
            pl.BlockSpec((_N_OPS, 1), lambda i: (0, 0)),
        ],
        out_shape=[
            jax.ShapeDtypeStruct((_NR, _NC), jnp.int32),
            jax.ShapeDtypeStruct((_N_OPS, 1), jnp.float32),
        ],
    )(pa, pb, mcr, arr, acc, obsr, obsc)


def kernel(samples, theta, kernel, observables):
    samples_t = samples.T
    codes_a = _codes_stage(samples_t, 0)
    codes_b = _codes_stage(samples_t, _GRID_H)
    pa = _sc_hist(codes_a)
    pb = _sc_hist(codes_b)

    a = kernel + jnp.diag(theta)
    nb, cb = _NB, _CB
    mcr = jnp.zeros((nb, nb), jnp.float32).at[:_RB, :cb].set(
        a[cb:, :cb] + a[:cb, cb:].T)
    arr = jnp.zeros((nb, nb), jnp.float32).at[:_RB, :_RB].set(a[cb:, cb:])
    acc = jnp.zeros((nb, nb), jnp.float32).at[:cb, :cb].set(a[:cb, :cb])
    obsr = jnp.zeros((_N_OPS, nb), jnp.float32).at[:, :_RB].set(
        observables[:, cb:])
    obsc = jnp.zeros((_N_OPS, nb), jnp.float32).at[:, :cb].set(
        observables[:, :cb])

    counts2d, exps = _bins_stage(pa.reshape(_N_CORES, _NR, _NC),
                                 pb.reshape(_N_CORES, _NR, _NC),
                                 mcr, arr, acc, obsr, obsc)
    return counts2d.reshape(_NUM_BINS), exps.reshape(_N_OPS)
